# baseline (speedup 1.0000x reference)
import functools

import jax
import jax.numpy as jnp
from jax import lax
from jax.experimental import pallas as pl
from jax.experimental.pallas import tpu as pltpu
from jax.experimental.pallas import tpu_sc as plsc

F = 26
CARD = 1000
D = 128
B = 4096
L = 16
NC, NS = 2, 16
FPC = F // NC
ROWS = B * F
BPT = B // NS
CHUNK = 128
CPF = BPT // CHUNK
VPR = D // L
NB = 4
STG = CARD // NS + (CARD % NS > 0)
SBIG = 64
STAIL = CARD - (NS - 1) * SBIG

_mesh = plsc.VectorSubcoreMesh(core_axis_name="c", subcore_axis_name="s")


@functools.partial(
    pl.kernel,
    out_type=jax.ShapeDtypeStruct((ROWS, D), jnp.float32),
    mesh=_mesh,
    scratch_types=[
        pltpu.VMEM_SHARED((2 * CARD, D), jnp.float32),
        pltpu.VMEM((BPT,), jnp.int32),
        pltpu.VMEM((F, D), jnp.float32),
        pltpu.SemaphoreType.DMA,
    ]
    + [pltpu.VMEM((CHUNK, D), jnp.float32) for _ in range(NB)]
    + [pltpu.SemaphoreType.DMA for _ in range(2 * NB)],
)
def _tokenize(idx_hbm, tab_hbm, bias_hbm, out_hbm, shr_v, gid_v, bias_v, sem_s,
              *bufs_sems):
    bufq = bufs_sems[:NB]
    sem_g = bufs_sems[NB:2 * NB]
    sem_w = bufs_sems[2 * NB:]
    cid = lax.axis_index("c")
    sid = lax.axis_index("s")
    f0 = cid * FPC
    lane = lax.iota(jnp.int32, L)

    pltpu.sync_copy(bias_hbm, bias_v)

    def stage(i):
        tbase = (f0 + i) * CARD
        slot = (i % 2) * CARD

        @pl.when(sid < NS - 1)
        def _main():
            off = sid * SBIG
            pltpu.async_copy(
                tab_hbm.at[pl.ds(tbase + off, SBIG)],
                shr_v.at[pl.ds(slot + off, SBIG)], sem_s,
            )

        @pl.when(sid == NS - 1)
        def _tail():
            off = (NS - 1) * SBIG
            pltpu.async_copy(
                tab_hbm.at[pl.ds(tbase + off, STAIL)],
                shr_v.at[pl.ds(slot + off, STAIL)], sem_s,
            )

    def wait_stage():
        @pl.when(sid < NS - 1)
        def _main():
            pltpu.make_async_copy(
                tab_hbm.at[pl.ds(0, SBIG)], shr_v.at[pl.ds(0, SBIG)], sem_s
            ).wait()

        @pl.when(sid == NS - 1)
        def _tail():
            pltpu.make_async_copy(
                tab_hbm.at[pl.ds(0, STAIL)], shr_v.at[pl.ds(0, STAIL)], sem_s
            ).wait()

    stage(0)
    wait_stage()
    plsc.subcore_barrier()

    wd = {}
    for i in range(FPC):
        if i + 1 < FPC:
            stage(i + 1)
        slot = (i % 2) * CARD
        obase = (f0 + i) * B + sid * BPT
        pltpu.sync_copy(idx_hbm.at[pl.ds(obase, BPT)], gid_v)
        for g in range(BPT // L):
            sl = pl.ds(g * L, L)
            gid_v[sl] = jnp.maximum(gid_v[sl], 0) + slot

        gds = []
        for h in range(CPF):
            k = i * CPF + h
            if k - NB >= 0:
                wd[k - NB].wait()
            gds.append(pltpu.async_copy(
                shr_v.at[gid_v.at[pl.ds(h * CHUNK, CHUNK)]],
                bufq[k % NB], sem_g[k % NB],
            ))
        fk = f0 + i
        bvals = [bias_v[fk, pl.ds(j * L, L)] for j in range(VPR)]
        for h in range(CPF):
            k = i * CPF + h
            s = k % NB
            gds[h].wait()
            buf = bufq[s]

            @pl.loop(0, CHUNK, unroll=2)
            def _bias_add(r):
                for j in range(VPR):
                    sl = pl.ds(j * L, L)
                    buf[r, sl] = buf[r, sl] + bvals[j]

            wd[k] = pltpu.async_copy(
                buf, out_hbm.at[pl.ds(obase + h * CHUNK, CHUNK), :], sem_w[s]
            )

        if i + 1 < FPC:
            wait_stage()
        plsc.subcore_barrier()

    for k in range(max(0, FPC * CPF - NB), FPC * CPF):
        wd[k].wait()


def kernel(x_cat, tables, bias):
    idx_fmajor = x_cat.astype(jnp.int32).T.reshape(ROWS)
    tab = tables.reshape(F * CARD, D)
    out = _tokenize(idx_fmajor, tab, bias)
    return out.reshape(F, B, D).transpose(1, 0, 2)


# speedup vs baseline: 1.1222x; 1.1222x over previous
import functools

import jax
import jax.numpy as jnp
from jax import lax
from jax.experimental import pallas as pl
from jax.experimental.pallas import tpu as pltpu
from jax.experimental.pallas import tpu_sc as plsc

F = 26
CARD = 1000
D = 128
B = 4096
L = 16
NC, NS = 2, 16
NW = NC * NS
ROWS = B * F
RPW = ROWS // NW
CHUNK = 128
NCH = RPW // CHUNK
VPR = D // L
NB = 6
DEPTH = 4

_mesh = plsc.VectorSubcoreMesh(core_axis_name="c", subcore_axis_name="s")


@functools.partial(
    pl.kernel,
    out_type=jax.ShapeDtypeStruct((ROWS, D), jnp.float32),
    mesh=_mesh,
    scratch_types=[
        pltpu.VMEM((RPW,), jnp.int32),
        pltpu.VMEM((F, D), jnp.float32),
    ]
    + [pltpu.VMEM((CHUNK, D), jnp.float32) for _ in range(NB)]
    + [pltpu.SemaphoreType.DMA for _ in range(2 * NB)],
)
def _tokenize(idx_hbm, tab_hbm, bias_hbm, out_hbm, gid_v, bias_v, *bufs_sems):
    bufq = bufs_sems[:NB]
    sem_g = bufs_sems[NB:2 * NB]
    sem_w = bufs_sems[2 * NB:]
    wid = lax.axis_index("s") * NC + lax.axis_index("c")
    wbase = wid * RPW
    lane = lax.iota(jnp.int32, L)

    pltpu.sync_copy(idx_hbm.at[pl.ds(wbase, RPW)], gid_v)
    pltpu.sync_copy(bias_hbm, bias_v)
    for g in range(RPW // L):
        sl = pl.ds(g * L, L)
        fvec = lax.shift_right_logical(wbase + g * L + lane, 12)
        gid_v[sl] = jnp.maximum(gid_v[sl], 0) + fvec * CARD

    gd, wd = {}, {}
    waited = set()

    def fire(k):
        gd[k] = pltpu.async_copy(
            tab_hbm.at[gid_v.at[pl.ds(k * CHUNK, CHUNK)]], bufq[k % NB], sem_g[k % NB]
        )

    for k in range(DEPTH - 1):
        fire(k)
    for k in range(NCH):
        s = k % NB
        if k + DEPTH - 1 < NCH:
            c = k - (NB - DEPTH)
            if c >= 0:
                wd[c].wait()
                waited.add(c)
            fire(k + DEPTH - 1)
        gd[k].wait()
        buf = bufq[s]
        fk = lax.shift_right_logical(wbase + k * CHUNK, 12)
        bvals = [bias_v[fk, pl.ds(j * L, L)] for j in range(VPR)]

        @pl.loop(0, CHUNK, unroll=2)
        def _bias_add(r):
            for j in range(VPR):
                sl = pl.ds(j * L, L)
                buf[r, sl] = buf[r, sl] + bvals[j]

        wd[k] = pltpu.async_copy(
            buf, out_hbm.at[pl.ds(wbase + k * CHUNK, CHUNK), :], sem_w[s]
        )

    for k in range(NCH):
        if k not in waited:
            wd[k].wait()


def kernel(x_cat, tables, bias):
    idx_fmajor = x_cat.astype(jnp.int32).T.reshape(ROWS)
    tab = tables.reshape(F * CARD, D)
    out = _tokenize(idx_fmajor, tab, bias)
    return out.reshape(F, B, D).transpose(1, 0, 2)
